# Initial kernel scaffold; baseline (speedup 1.0000x reference)
import jax
import jax.numpy as jnp
from jax.experimental import pallas as pl


def kernel(points, full_coors, coors_inv, coors, batch_size, W1, b1, W2, b2):
    raise NotImplementedError("write your pallas kernel here")


# speedup vs baseline: 1.3127x; 1.3127x over previous
import functools
import numpy as np
import jax
import jax.numpy as jnp
from jax import lax
from jax.experimental import pallas as pl
from jax.experimental.pallas import tpu as pltpu
from jax.experimental.pallas import tpu_sc as plsc

N = 400000
NUM_VOX = 60000
OUT_CH = 64
HALF = NUM_VOX // 2
DUMP = HALF
MB = 128
NMB = N // MB
NSUB = 16
NCORE = 2

_COORS_RANGE = np.array([[0.0, 50.0],
                         [-np.pi, np.pi],
                         [-4.0, 2.0]], dtype=np.float32)
_SPATIAL = np.array([480.0, 360.0, 32.0], dtype=np.float32)
_INTERVALS = (_COORS_RANGE[:, 1] - _COORS_RANGE[:, 0]) / _SPATIAL
_MINS = _COORS_RANGE[:, 0]


def _ka_body(pts_hbm, inv_hbm, zeros_hbm, out_hbm, pbuf, ibuf, sem, T):
    c = lax.axis_index("c")
    s = lax.axis_index("s")
    w = s * NCORE + c

    stripe = 3752
    last = NUM_VOX - 15 * stripe
    @pl.when(s < 15)
    def _():
        pltpu.sync_copy(zeros_hbm, T.at[pl.ds(s * stripe, stripe)])
    @pl.when(s == 15)
    def _():
        pltpu.sync_copy(zeros_hbm.at[pl.ds(0, last)],
                        T.at[pl.ds(15 * stripe, last)])
    plsc.subcore_barrier()

    n_iters = NMB // 32 + 1

    def body(t, carry):
        m = w + t * 32
        @pl.when(m < NMB)
        def _():
            base = m * MB
            pltpu.async_copy(pts_hbm.at[pl.ds(base, MB)], pbuf, sem).wait()
            pltpu.sync_copy(inv_hbm.at[pl.ds(base, MB)], ibuf)
            pltpu.sync_copy(pbuf, T.at[ibuf], add=True)
        return carry

    lax.fori_loop(0, n_iters, body, 0)
    plsc.subcore_barrier()
    @pl.when(s < 15)
    def _():
        pltpu.sync_copy(T.at[pl.ds(s * stripe, stripe)],
                        out_hbm.at[c, pl.ds(s * stripe, stripe)])
    @pl.when(s == 15)
    def _():
        pltpu.sync_copy(T.at[pl.ds(15 * stripe, last)],
                        out_hbm.at[c, pl.ds(15 * stripe, last)])


def _run_ka(points, coors_inv):
    zeros = jnp.zeros((3752, 16), jnp.float32)
    mesh = plsc.VectorSubcoreMesh(core_axis_name="c", subcore_axis_name="s")
    return pl.kernel(
        _ka_body,
        out_type=jax.ShapeDtypeStruct((NCORE, NUM_VOX, 16), jnp.float32),
        scratch_types=[
            pltpu.VMEM((MB, 16), jnp.float32),
            pltpu.VMEM((MB,), jnp.int32),
            pltpu.SemaphoreType.DMA,
            pltpu.VMEM_SHARED((NUM_VOX, 16), jnp.float32),
        ],
        mesh=mesh,
        compiler_params=pltpu.CompilerParams(use_tc_tiling_on_sc=False),
    )(points, coors_inv, zeros)


_PB = 2000


def _kb_body(pts_ref, fc_ref, w1_ref, b1_ref, out_ref, vals_ref):
    pts = pts_ref[...]
    x = pts[:, 0:1]
    y = pts[:, 1:2]
    z = pts[:, 2:3]
    feat = pts[:, 3:4]
    rho = jnp.sqrt(x * x + y * y)
    phi = jnp.arctan2(y, x)
    grid = fc_ref[...][:, 1:4].astype(jnp.float32)
    c0 = grid[:, 0:1] * float(_INTERVALS[0]) + float(_MINS[0] + 0.5 * _INTERVALS[0])
    c1 = grid[:, 1:2] * float(_INTERVALS[1]) + float(_MINS[1] + 0.5 * _INTERVALS[1])
    c2 = grid[:, 2:3] * float(_INTERVALS[2]) + float(_MINS[2] + 0.5 * _INTERVALS[2])
    F = jnp.concatenate(
        [feat, rho, phi, z, x, y, z, rho - c0, phi - c1, z - c2],
        axis=1)
    out_ref[...] = jnp.dot(F, w1_ref[...],
                           preferred_element_type=jnp.float32) + b1_ref[...]
    zero = jnp.zeros_like(x)
    vals_ref[...] = jnp.concatenate(
        [x, y, z, jnp.ones_like(x)] + [zero] * 12, axis=1)


def _run_kb(points, full_coors, W1, b1):
    grid = N // _PB
    return pl.pallas_call(
        _kb_body,
        grid=(grid,),
        in_specs=[
            pl.BlockSpec((_PB, 4), lambda i: (i, 0)),
            pl.BlockSpec((_PB, 4), lambda i: (i, 0)),
            pl.BlockSpec((10, OUT_CH), lambda i: (0, 0)),
            pl.BlockSpec((1, OUT_CH), lambda i: (0, 0)),
        ],
        out_specs=[
            pl.BlockSpec((_PB, OUT_CH), lambda i: (i, 0)),
            pl.BlockSpec((_PB, 16), lambda i: (i, 0)),
        ],
        out_shape=[
            jax.ShapeDtypeStruct((N, OUT_CH), jnp.float32),
            jax.ShapeDtypeStruct((N, 16), jnp.float32),
        ],
    )(points, full_coors, W1, b1.reshape(1, OUT_CH))


_VB = 6000


def _kc_body(t0_ref, t1_ref, w1s_ref, c_ref, invd_ref, fact_ref):
    T = t0_ref[...] + t1_ref[...]
    cnt = T[:, 3:4]
    denom = jnp.maximum(cnt, 1.0)
    pc_mean = T[:, 0:3] / denom
    c_ref[...] = jnp.dot(pc_mean, w1s_ref[...],
                         preferred_element_type=jnp.float32)
    invd_ref[...] = 1.0 / denom
    fact_ref[...] = jnp.minimum(cnt, 1.0)


def _run_kc(t0, t1, W1sub):
    grid = NUM_VOX // _VB
    return pl.pallas_call(
        _kc_body,
        grid=(grid,),
        in_specs=[
            pl.BlockSpec((_VB, 16), lambda i: (i, 0)),
            pl.BlockSpec((_VB, 16), lambda i: (i, 0)),
            pl.BlockSpec((3, OUT_CH), lambda i: (0, 0)),
        ],
        out_specs=[
            pl.BlockSpec((_VB, OUT_CH), lambda i: (i, 0)),
            pl.BlockSpec((_VB, 1), lambda i: (i, 0)),
            pl.BlockSpec((_VB, 1), lambda i: (i, 0)),
        ],
        out_shape=[
            jax.ShapeDtypeStruct((NUM_VOX, OUT_CH), jnp.float32),
            jax.ShapeDtypeStruct((NUM_VOX, 1), jnp.float32),
            jax.ShapeDtypeStruct((NUM_VOX, 1), jnp.float32),
        ],
    )(t0, t1, W1sub)


QUART = NUM_VOX // 4


def _kd_body(g0_hbm, inv_hbm, c_hbm, zeros_hbm, out_hbm,
             idxb, idxloc, cbuf, vbuf, g0b, sem, T):
    c = lax.axis_index("c")
    s = lax.axis_index("s")

    stripe = 944
    last = QUART - 15 * stripe
    n_iters = NMB // NSUB + 1

    for p in range(2):
        lo_v = (2 * c + p) * QUART
        hi_v = lo_v + QUART

        @pl.when(s < 15)
        def _():
            pltpu.sync_copy(zeros_hbm.at[pl.ds(0, stripe)],
                            T.at[pl.ds(s * stripe, stripe)])
        @pl.when(s == 15)
        def _():
            pltpu.sync_copy(zeros_hbm.at[pl.ds(0, last)],
                            T.at[pl.ds(15 * stripe, last)])
        @pl.when(s == 0)
        def _():
            pltpu.sync_copy(zeros_hbm.at[pl.ds(0, 8)], T.at[pl.ds(QUART, 8)])
        plsc.subcore_barrier()

        def body(t, carry):
            m = s + t * NSUB
            @pl.when(m < NMB)
            def _():
                base = m * MB
                pltpu.sync_copy(inv_hbm.at[pl.ds(base, MB)], idxb)
                gather = pltpu.async_copy(c_hbm.at[idxb], cbuf, sem)
                pltpu.sync_copy(
                    g0_hbm.at[pl.ds(base * OUT_CH, MB * OUT_CH)], g0b)
                for j in range(MB // 16):
                    v = idxb[pl.ds(j * 16, 16)] - lo_v
                    ok = (v >= 0) & (v < QUART)
                    idxloc[pl.ds(j * 16, 16)] = jnp.where(ok, v, QUART)
                gather.wait()

                def ew(r, carry2):
                    for q in range(OUT_CH // 16):
                        g = g0b[pl.ds(r * OUT_CH + q * 16, 16)]
                        cc = cbuf[r, pl.ds(q * 16, 16)]
                        vbuf[r, pl.ds(q * 16, 16)] = jnp.maximum(g - cc, 0.0)
                    return carry2

                lax.fori_loop(0, MB, ew, 0)
                pltpu.sync_copy(vbuf, T.at[idxloc], add=True)
            return carry

        lax.fori_loop(0, n_iters, body, 0)
        plsc.subcore_barrier()
        @pl.when(s < 15)
        def _():
            pltpu.sync_copy(T.at[pl.ds(s * stripe, stripe)],
                            out_hbm.at[pl.ds(lo_v + s * stripe, stripe)])
        @pl.when(s == 15)
        def _():
            pltpu.sync_copy(T.at[pl.ds(15 * stripe, last)],
                            out_hbm.at[pl.ds(lo_v + 15 * stripe, last)])
        plsc.subcore_barrier()


def _run_kd(g0_flat, coors_inv, cmat):
    zeros = jnp.zeros((944, OUT_CH), jnp.float32)
    mesh = plsc.VectorSubcoreMesh(core_axis_name="c", subcore_axis_name="s")
    return pl.kernel(
        _kd_body,
        out_type=jax.ShapeDtypeStruct((NUM_VOX, OUT_CH), jnp.float32),
        scratch_types=[
            pltpu.VMEM((MB,), jnp.int32),
            pltpu.VMEM((MB,), jnp.int32),
            pltpu.VMEM((MB, OUT_CH), jnp.float32),
            pltpu.VMEM((MB, OUT_CH), jnp.float32),
            pltpu.VMEM((MB * OUT_CH,), jnp.float32),
            pltpu.SemaphoreType.DMA,
            pltpu.VMEM_SHARED((QUART + 8, OUT_CH), jnp.float32),
        ],
        mesh=mesh,
        compiler_params=pltpu.CompilerParams(use_tc_tiling_on_sc=False),
    )(g0_flat, coors_inv, cmat, zeros)


def _ke_body(h_ref, invd_ref, fact_ref, w2_ref, b2_ref, out_ref):
    hm = h_ref[...] * invd_ref[...]
    out_ref[...] = (jnp.dot(hm, w2_ref[...], preferred_element_type=jnp.float32)
                    + b2_ref[...] * fact_ref[...])


def _run_ke(H, invd, fact, W2, b2):
    grid = NUM_VOX // _VB
    return pl.pallas_call(
        _ke_body,
        grid=(grid,),
        in_specs=[
            pl.BlockSpec((_VB, OUT_CH), lambda i: (i, 0)),
            pl.BlockSpec((_VB, 1), lambda i: (i, 0)),
            pl.BlockSpec((_VB, 1), lambda i: (i, 0)),
            pl.BlockSpec((OUT_CH, OUT_CH), lambda i: (0, 0)),
            pl.BlockSpec((1, OUT_CH), lambda i: (0, 0)),
        ],
        out_specs=pl.BlockSpec((_VB, OUT_CH), lambda i: (i, 0)),
        out_shape=jax.ShapeDtypeStruct((NUM_VOX, OUT_CH), jnp.float32),
    )(H, invd, fact, W2, b2.reshape(1, OUT_CH))


@jax.jit
def _impl(points, full_coors, coors_inv, W1, b1, W2, b2):
    g0, vals4 = _run_kb(points, full_coors, W1, b1)
    partials = _run_ka(vals4, coors_inv)
    cmat, invd, fact = _run_kc(partials[0], partials[1], W1[4:7])
    H = _run_kd(g0.reshape(-1), coors_inv, cmat)
    return _run_ke(H, invd, fact, W2, b2)


def kernel(points, full_coors, coors_inv, coors, batch_size, W1, b1, W2, b2):
    del coors, batch_size
    return _impl(points, full_coors, coors_inv.astype(jnp.int32), W1, b1, W2, b2)


# speedup vs baseline: 1.3331x; 1.0156x over previous
import functools
import numpy as np
import jax
import jax.numpy as jnp
from jax import lax
from jax.experimental import pallas as pl
from jax.experimental.pallas import tpu as pltpu
from jax.experimental.pallas import tpu_sc as plsc

N = 400000
NUM_VOX = 60000
OUT_CH = 64
HALF = NUM_VOX // 2
DUMP = HALF
MB = 128
NMB = N // MB
NSUB = 16
NCORE = 2

_COORS_RANGE = np.array([[0.0, 50.0],
                         [-np.pi, np.pi],
                         [-4.0, 2.0]], dtype=np.float32)
_SPATIAL = np.array([480.0, 360.0, 32.0], dtype=np.float32)
_INTERVALS = (_COORS_RANGE[:, 1] - _COORS_RANGE[:, 0]) / _SPATIAL
_MINS = _COORS_RANGE[:, 0]


def _ka_body(pts_hbm, inv_hbm, zeros_hbm, out_hbm, pbuf, ibuf, sem, T):
    c = lax.axis_index("c")
    s = lax.axis_index("s")
    w = s * NCORE + c

    stripe = 3752
    last = NUM_VOX - 15 * stripe
    @pl.when(s < 15)
    def _():
        pltpu.sync_copy(zeros_hbm, T.at[pl.ds(s * stripe, stripe)])
    @pl.when(s == 15)
    def _():
        pltpu.sync_copy(zeros_hbm.at[pl.ds(0, last)],
                        T.at[pl.ds(15 * stripe, last)])
    plsc.subcore_barrier()

    n_iters = NMB // 32 + 1

    def body(t, carry):
        m = w + t * 32
        @pl.when(m < NMB)
        def _():
            base = m * MB
            pltpu.async_copy(pts_hbm.at[pl.ds(base, MB)], pbuf, sem).wait()
            pltpu.sync_copy(inv_hbm.at[pl.ds(base, MB)], ibuf)
            pltpu.sync_copy(pbuf, T.at[ibuf], add=True)
        return carry

    lax.fori_loop(0, n_iters, body, 0)
    plsc.subcore_barrier()
    @pl.when(s < 15)
    def _():
        pltpu.sync_copy(T.at[pl.ds(s * stripe, stripe)],
                        out_hbm.at[c, pl.ds(s * stripe, stripe)])
    @pl.when(s == 15)
    def _():
        pltpu.sync_copy(T.at[pl.ds(15 * stripe, last)],
                        out_hbm.at[c, pl.ds(15 * stripe, last)])


def _run_ka(points, coors_inv):
    zeros = jnp.zeros((3752, 16), jnp.float32)
    mesh = plsc.VectorSubcoreMesh(core_axis_name="c", subcore_axis_name="s")
    return pl.kernel(
        _ka_body,
        out_type=jax.ShapeDtypeStruct((NCORE, NUM_VOX, 16), jnp.float32),
        scratch_types=[
            pltpu.VMEM((MB, 16), jnp.float32),
            pltpu.VMEM((MB,), jnp.int32),
            pltpu.SemaphoreType.DMA,
            pltpu.VMEM_SHARED((NUM_VOX, 16), jnp.float32),
        ],
        mesh=mesh,
        compiler_params=pltpu.CompilerParams(use_tc_tiling_on_sc=False),
    )(points, coors_inv, zeros)


_PB = 2000


def _kb_body(pts_ref, fc_ref, w1_ref, b1_ref, out_ref, outb_ref, vals_ref):
    pts = pts_ref[...]
    x = pts[:, 0:1]
    y = pts[:, 1:2]
    z = pts[:, 2:3]
    feat = pts[:, 3:4]
    rho = jnp.sqrt(x * x + y * y)
    phi = jnp.arctan2(y, x)
    grid = fc_ref[...][:, 1:4].astype(jnp.float32)
    c0 = grid[:, 0:1] * float(_INTERVALS[0]) + float(_MINS[0] + 0.5 * _INTERVALS[0])
    c1 = grid[:, 1:2] * float(_INTERVALS[1]) + float(_MINS[1] + 0.5 * _INTERVALS[1])
    c2 = grid[:, 2:3] * float(_INTERVALS[2]) + float(_MINS[2] + 0.5 * _INTERVALS[2])
    F = jnp.concatenate(
        [feat, rho, phi, z, x, y, z, rho - c0, phi - c1, z - c2],
        axis=1)
    g = jnp.dot(F, w1_ref[...],
                preferred_element_type=jnp.float32) + b1_ref[...]
    out_ref[...] = g[:, :OUT_CH // 2]
    outb_ref[...] = g[:, OUT_CH // 2:]
    zero = jnp.zeros_like(x)
    vals_ref[...] = jnp.concatenate(
        [x, y, z, jnp.ones_like(x)] + [zero] * 12, axis=1)


def _run_kb(points, full_coors, W1, b1):
    grid = N // _PB
    return pl.pallas_call(
        _kb_body,
        grid=(grid,),
        in_specs=[
            pl.BlockSpec((_PB, 4), lambda i: (i, 0)),
            pl.BlockSpec((_PB, 4), lambda i: (i, 0)),
            pl.BlockSpec((10, OUT_CH), lambda i: (0, 0)),
            pl.BlockSpec((1, OUT_CH), lambda i: (0, 0)),
        ],
        out_specs=[
            pl.BlockSpec((_PB, OUT_CH // 2), lambda i: (i, 0)),
            pl.BlockSpec((_PB, OUT_CH // 2), lambda i: (i, 0)),
            pl.BlockSpec((_PB, 16), lambda i: (i, 0)),
        ],
        out_shape=[
            jax.ShapeDtypeStruct((N, OUT_CH // 2), jnp.float32),
            jax.ShapeDtypeStruct((N, OUT_CH // 2), jnp.float32),
            jax.ShapeDtypeStruct((N, 16), jnp.float32),
        ],
    )(points, full_coors, W1, b1.reshape(1, OUT_CH))


_VB = 6000


def _kc_body(t0_ref, t1_ref, w1s_ref, c0_ref, c1_ref, invd_ref, fact_ref):
    T = t0_ref[...] + t1_ref[...]
    cnt = T[:, 3:4]
    denom = jnp.maximum(cnt, 1.0)
    pc_mean = T[:, 0:3] / denom
    cm = jnp.dot(pc_mean, w1s_ref[...], preferred_element_type=jnp.float32)
    c0_ref[...] = cm[:, :OUT_CH // 2]
    c1_ref[...] = cm[:, OUT_CH // 2:]
    invd_ref[...] = 1.0 / denom
    fact_ref[...] = jnp.minimum(cnt, 1.0)


def _run_kc(t0, t1, W1sub):
    grid = NUM_VOX // _VB
    return pl.pallas_call(
        _kc_body,
        grid=(grid,),
        in_specs=[
            pl.BlockSpec((_VB, 16), lambda i: (i, 0)),
            pl.BlockSpec((_VB, 16), lambda i: (i, 0)),
            pl.BlockSpec((3, OUT_CH), lambda i: (0, 0)),
        ],
        out_specs=[
            pl.BlockSpec((_VB, OUT_CH // 2), lambda i: (i, 0)),
            pl.BlockSpec((_VB, OUT_CH // 2), lambda i: (i, 0)),
            pl.BlockSpec((_VB, 1), lambda i: (i, 0)),
            pl.BlockSpec((_VB, 1), lambda i: (i, 0)),
        ],
        out_shape=[
            jax.ShapeDtypeStruct((NUM_VOX, OUT_CH // 2), jnp.float32),
            jax.ShapeDtypeStruct((NUM_VOX, OUT_CH // 2), jnp.float32),
            jax.ShapeDtypeStruct((NUM_VOX, 1), jnp.float32),
            jax.ShapeDtypeStruct((NUM_VOX, 1), jnp.float32),
        ],
    )(t0, t1, W1sub)


HCH = OUT_CH // 2
RHALF = NUM_VOX // 2


def _kd_body(g0a_hbm, g0b_hbm, inv_hbm, c0_hbm, c1_hbm, zeros_hbm, out_hbm,
             idxb, idxloc, cbuf, vbuf, g0b, sem, T):
    c = lax.axis_index("c")
    s = lax.axis_index("s")

    stripe = 1880
    last = RHALF - 15 * stripe
    n_iters = NMB // NSUB + 1

    def half(cm_hbm, g_hbm):
        for p in range(2):
            lo_v = p * RHALF

            @pl.when(s < 15)
            def _():
                pltpu.sync_copy(zeros_hbm.at[pl.ds(0, stripe)],
                                T.at[pl.ds(s * stripe, stripe)])
            @pl.when(s == 15)
            def _():
                pltpu.sync_copy(zeros_hbm.at[pl.ds(0, last)],
                                T.at[pl.ds(15 * stripe, last)])
            @pl.when(s == 0)
            def _():
                pltpu.sync_copy(zeros_hbm.at[pl.ds(0, 8)],
                                T.at[pl.ds(RHALF, 8)])
            plsc.subcore_barrier()

            def body(t, carry):
                m = s + t * NSUB
                @pl.when(m < NMB)
                def _():
                    base = m * MB
                    pltpu.sync_copy(inv_hbm.at[pl.ds(base, MB)], idxb)
                    gather = pltpu.async_copy(cm_hbm.at[idxb], cbuf, sem)
                    pltpu.sync_copy(g_hbm.at[pl.ds(base * HCH, MB * HCH)],
                                    g0b)
                    for j in range(MB // 16):
                        v = idxb[pl.ds(j * 16, 16)] - lo_v
                        ok = (v >= 0) & (v < RHALF)
                        idxloc[pl.ds(j * 16, 16)] = jnp.where(ok, v, RHALF)
                    gather.wait()

                    def ew(i, carry2):
                        r = i * 4
                        for dr in range(4):
                            for q in range(HCH // 16):
                                g = g0b[pl.ds((r + dr) * HCH + q * 16, 16)]
                                cc = cbuf[r + dr, pl.ds(q * 16, 16)]
                                vbuf[r + dr, pl.ds(q * 16, 16)] = (
                                    jnp.maximum(g - cc, 0.0))
                        return carry2

                    lax.fori_loop(0, MB // 4, ew, 0)
                    pltpu.sync_copy(vbuf, T.at[idxloc], add=True)
                return carry

            lax.fori_loop(0, n_iters, body, 0)
            plsc.subcore_barrier()
            @pl.when(s < 15)
            def _():
                pltpu.sync_copy(T.at[pl.ds(s * stripe, stripe)],
                                out_hbm.at[c, pl.ds(lo_v + s * stripe,
                                                    stripe)])
            @pl.when(s == 15)
            def _():
                pltpu.sync_copy(T.at[pl.ds(15 * stripe, last)],
                                out_hbm.at[c, pl.ds(lo_v + 15 * stripe,
                                                    last)])
            plsc.subcore_barrier()

    @pl.when(c == 0)
    def _():
        half(c0_hbm, g0a_hbm)
    @pl.when(c == 1)
    def _():
        half(c1_hbm, g0b_hbm)


def _run_kd(g0a_flat, g0b_flat, coors_inv, c0, c1):
    zeros = jnp.zeros((1880, HCH), jnp.float32)
    mesh = plsc.VectorSubcoreMesh(core_axis_name="c", subcore_axis_name="s")
    return pl.kernel(
        _kd_body,
        out_type=jax.ShapeDtypeStruct((NCORE, NUM_VOX, HCH), jnp.float32),
        scratch_types=[
            pltpu.VMEM((MB,), jnp.int32),
            pltpu.VMEM((MB,), jnp.int32),
            pltpu.VMEM((MB, HCH), jnp.float32),
            pltpu.VMEM((MB, HCH), jnp.float32),
            pltpu.VMEM((MB * HCH,), jnp.float32),
            pltpu.SemaphoreType.DMA,
            pltpu.VMEM_SHARED((RHALF + 8, HCH), jnp.float32),
        ],
        mesh=mesh,
        compiler_params=pltpu.CompilerParams(use_tc_tiling_on_sc=False),
    )(g0a_flat, g0b_flat, coors_inv, c0, c1, zeros)


def _ke_body(h0_ref, h1_ref, invd_ref, fact_ref, w2_ref, b2_ref, out_ref):
    invd = invd_ref[...]
    hm = jnp.concatenate([h0_ref[...] * invd, h1_ref[...] * invd], axis=1)
    out_ref[...] = (jnp.dot(hm, w2_ref[...], preferred_element_type=jnp.float32)
                    + b2_ref[...] * fact_ref[...])


def _run_ke(h0, h1, invd, fact, W2, b2):
    grid = NUM_VOX // _VB
    return pl.pallas_call(
        _ke_body,
        grid=(grid,),
        in_specs=[
            pl.BlockSpec((_VB, HCH), lambda i: (i, 0)),
            pl.BlockSpec((_VB, HCH), lambda i: (i, 0)),
            pl.BlockSpec((_VB, 1), lambda i: (i, 0)),
            pl.BlockSpec((_VB, 1), lambda i: (i, 0)),
            pl.BlockSpec((OUT_CH, OUT_CH), lambda i: (0, 0)),
            pl.BlockSpec((1, OUT_CH), lambda i: (0, 0)),
        ],
        out_specs=pl.BlockSpec((_VB, OUT_CH), lambda i: (i, 0)),
        out_shape=jax.ShapeDtypeStruct((NUM_VOX, OUT_CH), jnp.float32),
    )(h0, h1, invd, fact, W2, b2.reshape(1, OUT_CH))


@jax.jit
def _impl(points, full_coors, coors_inv, W1, b1, W2, b2):
    g0a, g0b, vals4 = _run_kb(points, full_coors, W1, b1)
    partials = _run_ka(vals4, coors_inv)
    c0, c1, invd, fact = _run_kc(partials[0], partials[1], W1[4:7])
    H = _run_kd(g0a.reshape(-1), g0b.reshape(-1), coors_inv, c0, c1)
    return _run_ke(H[0], H[1], invd, fact, W2, b2)


def kernel(points, full_coors, coors_inv, coors, batch_size, W1, b1, W2, b2):
    del coors, batch_size
    return _impl(points, full_coors, coors_inv.astype(jnp.int32), W1, b1, W2, b2)


# speedup vs baseline: 1.5613x; 1.1712x over previous
import functools
import numpy as np
import jax
import jax.numpy as jnp
from jax import lax
from jax.experimental import pallas as pl
from jax.experimental.pallas import tpu as pltpu
from jax.experimental.pallas import tpu_sc as plsc

N = 400000
NUM_VOX = 60000
OUT_CH = 64
HALF = NUM_VOX // 2
DUMP = HALF
MB = 128
NMB = N // MB
NSUB = 16
NCORE = 2

_COORS_RANGE = np.array([[0.0, 50.0],
                         [-np.pi, np.pi],
                         [-4.0, 2.0]], dtype=np.float32)
_SPATIAL = np.array([480.0, 360.0, 32.0], dtype=np.float32)
_INTERVALS = (_COORS_RANGE[:, 1] - _COORS_RANGE[:, 0]) / _SPATIAL
_MINS = _COORS_RANGE[:, 0]


def _ka_body(pts_hbm, inv_hbm, zeros_hbm, out_hbm, pbuf, ibuf, sem, T):
    c = lax.axis_index("c")
    s = lax.axis_index("s")
    w = s * NCORE + c

    stripe = 3752
    last = NUM_VOX - 15 * stripe
    @pl.when(s < 15)
    def _():
        pltpu.sync_copy(zeros_hbm, T.at[pl.ds(s * stripe, stripe)])
    @pl.when(s == 15)
    def _():
        pltpu.sync_copy(zeros_hbm.at[pl.ds(0, last)],
                        T.at[pl.ds(15 * stripe, last)])
    plsc.subcore_barrier()

    n_iters = NMB // 32 + 1

    def body(t, carry):
        m = w + t * 32
        @pl.when(m < NMB)
        def _():
            base = m * MB
            pltpu.async_copy(pts_hbm.at[pl.ds(base, MB)], pbuf, sem).wait()
            pltpu.sync_copy(inv_hbm.at[pl.ds(base, MB)], ibuf)
            pltpu.sync_copy(pbuf, T.at[ibuf], add=True)
        return carry

    lax.fori_loop(0, n_iters, body, 0)
    plsc.subcore_barrier()
    @pl.when(s < 15)
    def _():
        pltpu.sync_copy(T.at[pl.ds(s * stripe, stripe)],
                        out_hbm.at[c, pl.ds(s * stripe, stripe)])
    @pl.when(s == 15)
    def _():
        pltpu.sync_copy(T.at[pl.ds(15 * stripe, last)],
                        out_hbm.at[c, pl.ds(15 * stripe, last)])


def _run_ka(points, coors_inv):
    zeros = jnp.zeros((3752, 16), jnp.float32)
    mesh = plsc.VectorSubcoreMesh(core_axis_name="c", subcore_axis_name="s")
    return pl.kernel(
        _ka_body,
        out_type=jax.ShapeDtypeStruct((NCORE, NUM_VOX, 16), jnp.float32),
        scratch_types=[
            pltpu.VMEM((MB, 16), jnp.float32),
            pltpu.VMEM((MB,), jnp.int32),
            pltpu.SemaphoreType.DMA,
            pltpu.VMEM_SHARED((NUM_VOX, 16), jnp.float32),
        ],
        mesh=mesh,
        compiler_params=pltpu.CompilerParams(use_tc_tiling_on_sc=False),
    )(points, coors_inv, zeros)


_PB = 2000


def _kb_body(pts_ref, fc_ref, w1_ref, b1_ref, out_ref, outb_ref, vals_ref):
    pts = pts_ref[...]
    x = pts[:, 0:1]
    y = pts[:, 1:2]
    z = pts[:, 2:3]
    feat = pts[:, 3:4]
    rho = jnp.sqrt(x * x + y * y)
    phi = jnp.arctan2(y, x)
    grid = fc_ref[...][:, 1:4].astype(jnp.float32)
    c0 = grid[:, 0:1] * float(_INTERVALS[0]) + float(_MINS[0] + 0.5 * _INTERVALS[0])
    c1 = grid[:, 1:2] * float(_INTERVALS[1]) + float(_MINS[1] + 0.5 * _INTERVALS[1])
    c2 = grid[:, 2:3] * float(_INTERVALS[2]) + float(_MINS[2] + 0.5 * _INTERVALS[2])
    F = jnp.concatenate(
        [feat, rho, phi, z, x, y, z, rho - c0, phi - c1, z - c2],
        axis=1)
    g = jnp.dot(F, w1_ref[...],
                preferred_element_type=jnp.float32) + b1_ref[...]
    out_ref[...] = g[:, :OUT_CH // 2]
    outb_ref[...] = g[:, OUT_CH // 2:]
    zero = jnp.zeros_like(x)
    vals_ref[...] = jnp.concatenate(
        [x, y, z, jnp.ones_like(x)] + [zero] * 12, axis=1)


def _run_kb(points, full_coors, W1, b1):
    grid = N // _PB
    return pl.pallas_call(
        _kb_body,
        grid=(grid,),
        in_specs=[
            pl.BlockSpec((_PB, 4), lambda i: (i, 0)),
            pl.BlockSpec((_PB, 4), lambda i: (i, 0)),
            pl.BlockSpec((10, OUT_CH), lambda i: (0, 0)),
            pl.BlockSpec((1, OUT_CH), lambda i: (0, 0)),
        ],
        out_specs=[
            pl.BlockSpec((_PB, OUT_CH // 2), lambda i: (i, 0)),
            pl.BlockSpec((_PB, OUT_CH // 2), lambda i: (i, 0)),
            pl.BlockSpec((_PB, 16), lambda i: (i, 0)),
        ],
        out_shape=[
            jax.ShapeDtypeStruct((N, OUT_CH // 2), jnp.float32),
            jax.ShapeDtypeStruct((N, OUT_CH // 2), jnp.float32),
            jax.ShapeDtypeStruct((N, 16), jnp.float32),
        ],
    )(points, full_coors, W1, b1.reshape(1, OUT_CH))


_VB = 6000


def _kc_body(t0_ref, t1_ref, w1s_ref, c0_ref, c1_ref, invd_ref, fact_ref):
    T = t0_ref[...] + t1_ref[...]
    cnt = T[:, 3:4]
    denom = jnp.maximum(cnt, 1.0)
    pc_mean = T[:, 0:3] / denom
    cm = jnp.dot(pc_mean, w1s_ref[...], preferred_element_type=jnp.float32)
    c0_ref[...] = cm[:, :OUT_CH // 2]
    c1_ref[...] = cm[:, OUT_CH // 2:]
    invd_ref[...] = 1.0 / denom
    fact_ref[...] = jnp.minimum(cnt, 1.0)


def _run_kc(t0, t1, W1sub):
    grid = NUM_VOX // _VB
    return pl.pallas_call(
        _kc_body,
        grid=(grid,),
        in_specs=[
            pl.BlockSpec((_VB, 16), lambda i: (i, 0)),
            pl.BlockSpec((_VB, 16), lambda i: (i, 0)),
            pl.BlockSpec((3, OUT_CH), lambda i: (0, 0)),
        ],
        out_specs=[
            pl.BlockSpec((_VB, OUT_CH // 2), lambda i: (i, 0)),
            pl.BlockSpec((_VB, OUT_CH // 2), lambda i: (i, 0)),
            pl.BlockSpec((_VB, 1), lambda i: (i, 0)),
            pl.BlockSpec((_VB, 1), lambda i: (i, 0)),
        ],
        out_shape=[
            jax.ShapeDtypeStruct((NUM_VOX, OUT_CH // 2), jnp.float32),
            jax.ShapeDtypeStruct((NUM_VOX, OUT_CH // 2), jnp.float32),
            jax.ShapeDtypeStruct((NUM_VOX, 1), jnp.float32),
            jax.ShapeDtypeStruct((NUM_VOX, 1), jnp.float32),
        ],
    )(t0, t1, W1sub)


HCH = OUT_CH // 2
RHALF = NUM_VOX // 2
NBT = 196
NBL = NMB - 15 * NBT


def _kd_body(g0a_hbm, g0b_hbm, inv2d_hbm, c0_hbm, c1_hbm, zeros_hbm, out_hbm,
             idxbig, idxloc, cbuf, vbuf, g0b, gsem0, gsem1, ssem0, ssem1, T):
    c = lax.axis_index("c")
    s = lax.axis_index("s")
    gsem = (gsem0, gsem1)
    ssem = (ssem0, ssem1)

    stripe = 1880
    last = RHALF - 15 * stripe
    mbase = s * NBT
    nb = jnp.where(s < 15, NBT, NBL)

    @pl.when(s < 15)
    def _():
        pltpu.sync_copy(inv2d_hbm.at[pl.ds(mbase, NBT)], idxbig)
    @pl.when(s == 15)
    def _():
        pltpu.sync_copy(inv2d_hbm.at[pl.ds(mbase, NBL)],
                        idxbig.at[pl.ds(0, NBL)])

    def half(cm_hbm, g_hbm):
        for p in range(2):
            lo_v = p * RHALF

            @pl.when(s < 15)
            def _():
                pltpu.sync_copy(zeros_hbm.at[pl.ds(0, stripe)],
                                T.at[pl.ds(s * stripe, stripe)])
            @pl.when(s == 15)
            def _():
                pltpu.sync_copy(zeros_hbm.at[pl.ds(0, last)],
                                T.at[pl.ds(15 * stripe, last)])
            @pl.when(s == 0)
            def _():
                pltpu.sync_copy(zeros_hbm.at[pl.ds(0, 8)],
                                T.at[pl.ds(RHALF, 8)])
            plsc.subcore_barrier()

            pltpu.async_copy(cm_hbm.at[idxbig.at[0]], cbuf.at[0], gsem[0])

            def step(t, k):
                @pl.when(t < nb)
                def _():
                    kn = (k + 1) % 2
                    @pl.when(t + 1 < nb)
                    def _():
                        pltpu.async_copy(cm_hbm.at[idxbig.at[t + 1]],
                                         cbuf.at[kn], gsem[kn])
                    b = mbase + t
                    pltpu.sync_copy(
                        g_hbm.at[pl.ds(b * (MB * HCH), MB * HCH)], g0b)
                    for j in range(MB // 16):
                        v = idxbig[t, pl.ds(j * 16, 16)] - lo_v
                        ok = (v >= 0) & (v < RHALF)
                        idxloc[k, pl.ds(j * 16, 16)] = jnp.where(ok, v, RHALF)
                    pltpu.make_async_copy(cm_hbm.at[idxbig.at[t]],
                                          cbuf.at[k], gsem[k]).wait()
                    @pl.when(t >= 2)
                    def _():
                        pltpu.make_async_copy(vbuf.at[k], T.at[idxloc.at[k]],
                                              ssem[k]).wait()

                    def ew(i, carry2):
                        r = i * 4
                        for dr in range(4):
                            for q in range(HCH // 16):
                                g = g0b[pl.ds((r + dr) * HCH + q * 16, 16)]
                                cc = cbuf[k, r + dr, pl.ds(q * 16, 16)]
                                vbuf[k, r + dr, pl.ds(q * 16, 16)] = (
                                    jnp.maximum(g - cc, 0.0))
                        return carry2

                    lax.fori_loop(0, MB // 4, ew, 0)
                    pltpu.async_copy(vbuf.at[k], T.at[idxloc.at[k]],
                                     ssem[k], add=True)

            def pair(tt, carry):
                step(2 * tt, 0)
                step(2 * tt + 1, 1)
                return carry

            lax.fori_loop(0, (NBT + 1) // 2, pair, 0)
            for k in range(2):
                pltpu.make_async_copy(vbuf.at[k], T.at[idxloc.at[k]],
                                      ssem[k]).wait()
            plsc.subcore_barrier()
            @pl.when(s < 15)
            def _():
                pltpu.sync_copy(T.at[pl.ds(s * stripe, stripe)],
                                out_hbm.at[c, pl.ds(lo_v + s * stripe,
                                                    stripe)])
            @pl.when(s == 15)
            def _():
                pltpu.sync_copy(T.at[pl.ds(15 * stripe, last)],
                                out_hbm.at[c, pl.ds(lo_v + 15 * stripe,
                                                    last)])
            plsc.subcore_barrier()

    @pl.when(c == 0)
    def _():
        half(c0_hbm, g0a_hbm)
    @pl.when(c == 1)
    def _():
        half(c1_hbm, g0b_hbm)


def _run_kd(g0a_flat, g0b_flat, coors_inv2d, c0, c1):
    zeros = jnp.zeros((1880, HCH), jnp.float32)
    mesh = plsc.VectorSubcoreMesh(core_axis_name="c", subcore_axis_name="s")
    return pl.kernel(
        _kd_body,
        out_type=jax.ShapeDtypeStruct((NCORE, NUM_VOX, HCH), jnp.float32),
        scratch_types=[
            pltpu.VMEM((NBT, MB), jnp.int32),
            pltpu.VMEM((2, MB), jnp.int32),
            pltpu.VMEM((2, MB, HCH), jnp.float32),
            pltpu.VMEM((2, MB, HCH), jnp.float32),
            pltpu.VMEM((MB * HCH,), jnp.float32),
            pltpu.SemaphoreType.DMA,
            pltpu.SemaphoreType.DMA,
            pltpu.SemaphoreType.DMA,
            pltpu.SemaphoreType.DMA,
            pltpu.VMEM_SHARED((RHALF + 8, HCH), jnp.float32),
        ],
        mesh=mesh,
        compiler_params=pltpu.CompilerParams(use_tc_tiling_on_sc=False),
    )(g0a_flat, g0b_flat, coors_inv2d, c0, c1, zeros)


def _ke_body(h0_ref, h1_ref, invd_ref, fact_ref, w2_ref, b2_ref, out_ref):
    invd = invd_ref[...]
    hm = jnp.concatenate([h0_ref[...] * invd, h1_ref[...] * invd], axis=1)
    out_ref[...] = (jnp.dot(hm, w2_ref[...], preferred_element_type=jnp.float32)
                    + b2_ref[...] * fact_ref[...])


def _run_ke(h0, h1, invd, fact, W2, b2):
    grid = NUM_VOX // _VB
    return pl.pallas_call(
        _ke_body,
        grid=(grid,),
        in_specs=[
            pl.BlockSpec((_VB, HCH), lambda i: (i, 0)),
            pl.BlockSpec((_VB, HCH), lambda i: (i, 0)),
            pl.BlockSpec((_VB, 1), lambda i: (i, 0)),
            pl.BlockSpec((_VB, 1), lambda i: (i, 0)),
            pl.BlockSpec((OUT_CH, OUT_CH), lambda i: (0, 0)),
            pl.BlockSpec((1, OUT_CH), lambda i: (0, 0)),
        ],
        out_specs=pl.BlockSpec((_VB, OUT_CH), lambda i: (i, 0)),
        out_shape=jax.ShapeDtypeStruct((NUM_VOX, OUT_CH), jnp.float32),
    )(h0, h1, invd, fact, W2, b2.reshape(1, OUT_CH))


@jax.jit
def _impl(points, full_coors, coors_inv, W1, b1, W2, b2):
    g0a, g0b, vals4 = _run_kb(points, full_coors, W1, b1)
    partials = _run_ka(vals4, coors_inv)
    c0, c1, invd, fact = _run_kc(partials[0], partials[1], W1[4:7])
    H = _run_kd(g0a.reshape(-1), g0b.reshape(-1),
                coors_inv.reshape(NMB, MB), c0, c1)
    return _run_ke(H[0], H[1], invd, fact, W2, b2)


def kernel(points, full_coors, coors_inv, coors, batch_size, W1, b1, W2, b2):
    del coors, batch_size
    return _impl(points, full_coors, coors_inv.astype(jnp.int32), W1, b1, W2, b2)
